# baseline (speedup 1.0000x reference)
import jax
import jax.numpy as jnp
from jax.experimental import pallas as pl

_NUM_CLASSES = 1000
_CBLK = 40


def _onehot_body(x_ref, o_ref):
    f = pl.program_id(0)
    k = pl.program_id(1)
    row = x_ref[pl.ds(f, 1), :]
    cls = jax.lax.broadcasted_iota(jnp.int32, (_CBLK, 4096), 0) + k * _CBLK
    o_ref[0] = (cls == row).astype(jnp.float32)


def kernel(x):
    x = x.astype(jnp.int32)
    batch, feats = x.shape
    x_t = x.T
    out_phys = pl.pallas_call(
        _onehot_body,
        grid=(feats, _NUM_CLASSES // _CBLK),
        in_specs=[pl.BlockSpec((feats, batch), lambda f, k: (0, 0))],
        out_specs=pl.BlockSpec((1, _CBLK, batch), lambda f, k: (f, k, 0)),
        out_shape=jax.ShapeDtypeStruct((feats, _NUM_CLASSES, batch), jnp.float32),
    )(x_t)
    return out_phys.transpose(2, 0, 1)
